# Initial kernel scaffold; baseline (speedup 1.0000x reference)
import jax
import jax.numpy as jnp
from jax.experimental import pallas as pl


def kernel(point_cloud, params):
    raise NotImplementedError("write your pallas kernel here")


# speedup vs baseline: 1.3628x; 1.3628x over previous
import functools

import jax
import jax.numpy as jnp
import numpy as np
from jax import lax
from jax.experimental import pallas as pl
from jax.experimental.pallas import tpu as pltpu

B = 4
N = 2048
MLP = 32
RADIUS = [0.2, 0.4, 0.8]
NN_UP = [32, 32, 32]
NUM_SAMPLE = [4, 4, 4]
CHANNELS = [[64, 64], [128, 128], [128, 128]]
KERNEL = (8, 2, 2)
BIN_SIZE = 33
G_KERNEL = (8, 2, 1)
G_BIN = 17
GLOBAL_RADIUS = 100.0


def _bgather(t, idx):
    return jax.vmap(lambda a, i: a[i])(t, idx)


def _bn(x, gamma, beta, axes):
    m = x.mean(axis=axes, keepdims=True)
    v = x.var(axis=axes, keepdims=True)
    return (x - m) / jnp.sqrt(v + 1e-5) * gamma + beta


def _build_graph(xyz, radius, K, num_sample):
    sq = jnp.sum(xyz * xyz, axis=-1)
    d2 = sq[:, :, None] + sq[:, None, :] - 2.0 * jnp.einsum('bnd,bmd->bnm', xyz, xyz)
    d2 = jnp.maximum(d2, 0.0)
    masked = jnp.where(d2 <= radius * radius, d2, jnp.inf)
    negvals, idx = lax.top_k(-masked, K)
    d2k = -negvals
    valid = jnp.isfinite(d2k)
    cnt = valid.sum(axis=-1).astype(jnp.int32)
    idx = jnp.where(valid, idx, 0)
    dst = jnp.sqrt(jnp.maximum(jnp.where(valid, d2k, 0.0), 1e-12))
    m = xyz.shape[1] // num_sample
    samp = jnp.broadcast_to(jnp.arange(m) * num_sample, (xyz.shape[0], m))
    return idx, cnt, dst, samp


def _spherical_kernel(database, query, nn_idx, nn_cnt, nn_dst, radius, kernel):
    nbr = _bgather(database, nn_idx)
    delta = nbr - query[:, :, None, :]
    dist = nn_dst
    nA, nP, nR = kernel
    azim = jnp.arctan2(delta[..., 1], delta[..., 0])
    a_bin = jnp.clip(jnp.floor((azim + np.pi) / (2.0 * np.pi) * nA), 0, nA - 1).astype(jnp.int32)
    pol = jnp.arccos(jnp.clip(delta[..., 2] / jnp.maximum(dist, 1e-9), -1.0, 1.0))
    p_bin = jnp.clip(jnp.floor(pol / np.pi * nP), 0, nP - 1).astype(jnp.int32)
    r_bin = jnp.clip(jnp.floor(dist / radius * nR), 0, nR - 1).astype(jnp.int32)
    bins = a_bin * (nP * nR) + p_bin * nR + r_bin
    bin_size = nA * nP * nR + 1
    bins = jnp.where(dist < 1e-4, bin_size - 1, bins)
    return bins


def _sep_conv(net, nn_idx, nn_cnt, filt_idx, p):
    nbr = _bgather(net, nn_idx)
    w = p['Wd'][filt_idx]
    Kn = nn_idx.shape[2]
    mask = (jnp.arange(Kn)[None, None, :] < nn_cnt[:, :, None]).astype(net.dtype)
    depth = jnp.einsum('bmkc,bmkcu,bmk->bmcu', nbr, w, mask)
    b, m = depth.shape[0], depth.shape[1]
    x = depth.reshape(b, m, -1) @ p['Wp']
    x = _bn(x, p['gamma'], p['beta'], (0, 1))
    return jax.nn.relu(x)


def _pool3d(net, idx, cnt):
    nbr = _bgather(net, idx)
    Kn = idx.shape[2]
    mask = jnp.arange(Kn)[None, None, :, None] < cnt[:, :, None, None]
    return jnp.max(jnp.where(mask, nbr, -jnp.inf), axis=2)


def _cls_body(y_ref, w1_ref, g1_ref, b1_ref, w2_ref, g2_ref, b2_ref,
              w3_ref, b3_ref, out_ref):
    y = y_ref[...]
    h = jnp.dot(y, w1_ref[...], preferred_element_type=jnp.float32)
    m = h.mean(axis=0, keepdims=True)
    v = ((h - m) ** 2).mean(axis=0, keepdims=True)
    h = (h - m) / jnp.sqrt(v + 1e-5) * g1_ref[...] + b1_ref[...]
    h = jnp.maximum(h, 0.0)
    h2 = jnp.dot(h, w2_ref[...], preferred_element_type=jnp.float32)
    m2 = h2.mean(axis=0, keepdims=True)
    v2 = ((h2 - m2) ** 2).mean(axis=0, keepdims=True)
    h2 = (h2 - m2) / jnp.sqrt(v2 + 1e-5) * g2_ref[...] + b2_ref[...]
    h2 = jnp.maximum(h2, 0.0)
    out_ref[...] = jnp.dot(h2, w3_ref[...], preferred_element_type=jnp.float32) + b3_ref[...]


def _cls_head(y, c):
    return pl.pallas_call(
        _cls_body,
        out_shape=jax.ShapeDtypeStruct((y.shape[0], c['W3'].shape[1]), jnp.float32),
    )(y, c['W1'], c['g1'][None, :], c['b1'][None, :],
      c['W2'], c['g2'][None, :], c['b2'][None, :],
      c['W3'], c['b3'][None, :])


def kernel(point_cloud, params):
    pc = jnp.transpose(point_cloud, (0, 2, 1))
    pc = pc - pc.mean(axis=1, keepdims=True)
    scale = jnp.sqrt(jnp.max(jnp.sum(pc * pc, axis=-1, keepdims=True), axis=1, keepdims=True))
    pc = pc / scale
    xyz = pc
    query = xyz.mean(axis=1, keepdims=True)
    net = jax.nn.relu(_bn(xyz @ params['fc1']['W'], params['fc1']['gamma'], params['fc1']['beta'], (0, 1)))
    global_feat = []
    index = 0
    for l in range(3):
        net = jnp.concatenate([net, xyz], axis=2)
        intra_idx, intra_cnt, intra_dst, indices = _build_graph(xyz, RADIUS[l], NN_UP[l], NUM_SAMPLE[l])
        filt_idx = _spherical_kernel(xyz, xyz, intra_idx, intra_cnt, intra_dst, RADIUS[l], KERNEL)
        for _ in CHANNELS[l]:
            net = _sep_conv(net, intra_idx, intra_cnt, filt_idx, params['convs'][index])
            index += 1
        xyz = _bgather(xyz, indices)
        intra_idx = _bgather(intra_idx, indices)
        intra_cnt = _bgather(intra_cnt, indices)
        net = _pool3d(net, intra_idx, intra_cnt)
        global_feat.append(jnp.max(net, axis=1, keepdims=True))
    nl = xyz.shape[1]
    nn_idx = jnp.broadcast_to(jnp.arange(nl)[None, None, :], (xyz.shape[0], 1, nl))
    nn_cnt = jnp.full((xyz.shape[0], 1), nl, dtype=jnp.int32)
    nn_dst = jnp.sqrt(jnp.maximum(jnp.sum((xyz - query) ** 2, axis=-1), 1e-12))[:, None, :]
    filt_idx = _spherical_kernel(xyz, query, nn_idx, nn_cnt, nn_dst, GLOBAL_RADIUS, G_KERNEL)
    net = _sep_conv(net, nn_idx, nn_cnt, filt_idx, params['gconv'])
    global_feat.append(net)
    y = jnp.concatenate(global_feat, axis=2).reshape(point_cloud.shape[0], -1)
    return _cls_head(y, params['cls'])


# speedup vs baseline: 1.8268x; 1.3405x over previous
import functools

import jax
import jax.numpy as jnp
import numpy as np
from jax import lax
from jax.experimental import pallas as pl
from jax.experimental.pallas import tpu as pltpu

B = 4
N = 2048
MLP = 32
RADIUS = [0.2, 0.4, 0.8]
NN_UP = [32, 32, 32]
NUM_SAMPLE = [4, 4, 4]
CHANNELS = [[64, 64], [128, 128], [128, 128]]
KERNEL = (8, 2, 2)
BIN_SIZE = 33
G_KERNEL = (8, 2, 1)
G_BIN = 17
GLOBAL_RADIUS = 100.0


def _bgather(t, idx):
    return jax.vmap(lambda a, i: a[i])(t, idx)


def _bn(x, gamma, beta, axes):
    m = x.mean(axis=axes, keepdims=True)
    v = x.var(axis=axes, keepdims=True)
    return (x - m) / jnp.sqrt(v + 1e-5) * gamma + beta


_GRAPH_MODE = "jax_highest"
_PALLAS_LAYERS = (0, 1, 2)


def _build_graph(xyz, radius, K, num_sample):
    sq = jnp.sum(xyz * xyz, axis=-1)
    d2 = sq[:, :, None] + sq[:, None, :] - 2.0 * jnp.einsum('bnd,bmd->bnm', xyz, xyz)
    d2 = jnp.maximum(d2, 0.0)
    masked = jnp.where(d2 <= radius * radius, d2, jnp.inf)
    negvals, idx = lax.top_k(-masked, K)
    d2k = -negvals
    valid = jnp.isfinite(d2k)
    cnt = valid.sum(axis=-1).astype(jnp.int32)
    idx = jnp.where(valid, idx, 0)
    dst = jnp.sqrt(jnp.maximum(jnp.where(valid, d2k, 0.0), 1e-12))
    m = xyz.shape[1] // num_sample
    samp = jnp.broadcast_to(jnp.arange(m) * num_sample, (xyz.shape[0], m))
    return idx, cnt, dst, samp


def _spherical_kernel(database, query, nn_idx, nn_cnt, nn_dst, radius, kernel):
    nbr = _bgather(database, nn_idx)
    delta = nbr - query[:, :, None, :]
    dist = nn_dst
    nA, nP, nR = kernel
    azim = jnp.arctan2(delta[..., 1], delta[..., 0])
    a_bin = jnp.clip(jnp.floor((azim + np.pi) / (2.0 * np.pi) * nA), 0, nA - 1).astype(jnp.int32)
    pol = jnp.arccos(jnp.clip(delta[..., 2] / jnp.maximum(dist, 1e-9), -1.0, 1.0))
    p_bin = jnp.clip(jnp.floor(pol / np.pi * nP), 0, nP - 1).astype(jnp.int32)
    r_bin = jnp.clip(jnp.floor(dist / radius * nR), 0, nR - 1).astype(jnp.int32)
    bins = a_bin * (nP * nR) + p_bin * nR + r_bin
    bin_size = nA * nP * nR + 1
    bins = jnp.where(dist < 1e-4, bin_size - 1, bins)
    return bins


def _sep_conv(net, nn_idx, nn_cnt, filt_idx, p):
    nbr = _bgather(net, nn_idx)
    w = p['Wd'][filt_idx]
    Kn = nn_idx.shape[2]
    mask = (jnp.arange(Kn)[None, None, :] < nn_cnt[:, :, None]).astype(net.dtype)
    depth = jnp.einsum('bmkc,bmkcu,bmk->bmcu', nbr, w, mask)
    b, m = depth.shape[0], depth.shape[1]
    x = depth.reshape(b, m, -1) @ p['Wp']
    x = _bn(x, p['gamma'], p['beta'], (0, 1))
    return jax.nn.relu(x)


def _pool3d(net, idx, cnt):
    nbr = _bgather(net, idx)
    Kn = idx.shape[2]
    mask = jnp.arange(Kn)[None, None, :, None] < cnt[:, :, None, None]
    return jnp.max(jnp.where(mask, nbr, -jnp.inf), axis=2)


_BIG = 1e30


def _graph_body(K, radius, nA, nP, nR, rows_ref, all_ref, brows_ref, ball_ref,
                sqrows_ref, sqall_ref, idx_ref, cnt_ref, dst_ref, bins_ref):
    rows = rows_ref[0]
    alls = all_ref[0]
    M = rows.shape[0]
    Np = alls.shape[0]
    sq_rows = sqrows_ref[0]
    sq_all = sqall_ref[0]
    dot = lax.dot_general(brows_ref[0], ball_ref[0],
                          (((1,), (1,)), ((), ())),
                          preferred_element_type=jnp.float32)
    d2 = jnp.maximum(sq_rows + sq_all - 2.0 * dot, 0.0)
    work = jnp.where(d2 <= radius * radius, d2, _BIG)
    iota_n = lax.broadcasted_iota(jnp.int32, (M, Np), 1)
    iota_k = lax.broadcasted_iota(jnp.int32, (M, K), 1)
    vals = jnp.zeros((M, K), jnp.float32)
    idxs = jnp.zeros((M, K), jnp.int32)
    dx = jnp.zeros((M, K), jnp.float32)
    dy = jnp.zeros((M, K), jnp.float32)
    dz = jnp.zeros((M, K), jnp.float32)
    ax_ = alls[:, 0:1].T
    ay_ = alls[:, 1:2].T
    az_ = alls[:, 2:3].T
    for k in range(K):
        v = jnp.min(work, axis=1, keepdims=True)
        eq = work == v
        a = jnp.min(jnp.where(eq, iota_n, Np), axis=1, keepdims=True)
        oh = iota_n == a
        ohf = oh.astype(jnp.float32)
        nx = jnp.sum(ohf * ax_, axis=1, keepdims=True)
        ny = jnp.sum(ohf * ay_, axis=1, keepdims=True)
        nz = jnp.sum(ohf * az_, axis=1, keepdims=True)
        work = jnp.where(oh, _BIG, work)
        sel = iota_k == k
        vals = jnp.where(sel, v, vals)
        idxs = jnp.where(sel, a, idxs)
        dx = jnp.where(sel, nx - rows[:, 0:1], dx)
        dy = jnp.where(sel, ny - rows[:, 1:2], dy)
        dz = jnp.where(sel, nz - rows[:, 2:3], dz)
    valid = vals < (_BIG * 0.5)
    cnt_ref[0] = jnp.sum(valid.astype(jnp.int32), axis=1, keepdims=True)
    d2k = jnp.where(valid, vals, 0.0)
    dst = jnp.sqrt(jnp.maximum(d2k, 1e-12))
    dst_ref[0] = dst
    idx_ref[0] = jnp.where(valid, idxs, 0)
    ax, ay = jnp.abs(dx), jnp.abs(dy)
    neg = (
        jnp.where((dx < 0) & (ay >= ax), 1, 0)
        + jnp.where((dx >= 0) & (ay > ax), 2, 0)
        + jnp.where((dx > 0) & (ay <= ax), 3, 0)
    )
    pos = (
        jnp.where((dx > 0) & (ay < ax), 4, 0)
        + jnp.where((dx > 0) & (ay >= ax), 5, 0)
        + jnp.where((dx <= 0) & (ay > ax), 6, 0)
        + jnp.where((dx < 0) & (ay <= ax), 7, 0)
    )
    a_bin = jnp.where(dy < 0, neg, jnp.where((dx == 0) & (dy == 0), 4, pos))
    p_bin = jnp.where(dz > 0, 0, 1)
    r_bin = jnp.clip(jnp.floor(dst / radius * nR).astype(jnp.int32), 0, nR - 1)
    bins = a_bin * (nP * nR) + p_bin * nR + r_bin
    bins_ref[0] = jnp.where(dst < 1e-4, nA * nP * nR, bins)


def _graph_pallas(xyz, radius, K):
    b, n, _ = xyz.shape
    m = min(n, 256)
    nblk = n // m
    grid = (b, nblk)
    body = functools.partial(_graph_body, K, radius, 8, 2, 2)
    xb = xyz.astype(jnp.bfloat16)
    sq = jnp.sum(xyz * xyz, axis=-1)
    sq_c = sq[:, :, None]
    sq_r = sq[:, None, :]
    kvec = lambda: pl.BlockSpec((1, m, K), lambda bi, i: (bi, i, 0))
    idx, cnt, dst, bins = pl.pallas_call(
        body,
        grid=grid,
        in_specs=[
            pl.BlockSpec((1, m, 3), lambda bi, i: (bi, i, 0)),
            pl.BlockSpec((1, n, 3), lambda bi, i: (bi, 0, 0)),
            pl.BlockSpec((1, m, 3), lambda bi, i: (bi, i, 0)),
            pl.BlockSpec((1, n, 3), lambda bi, i: (bi, 0, 0)),
            pl.BlockSpec((1, m, 1), lambda bi, i: (bi, i, 0)),
            pl.BlockSpec((1, 1, n), lambda bi, i: (bi, 0, 0)),
        ],
        out_specs=[
            kvec(),
            pl.BlockSpec((1, m, 1), lambda bi, i: (bi, i, 0)),
            kvec(),
            kvec(),
        ],
        out_shape=[
            jax.ShapeDtypeStruct((b, n, K), jnp.int32),
            jax.ShapeDtypeStruct((b, n, 1), jnp.int32),
            jax.ShapeDtypeStruct((b, n, K), jnp.float32),
            jax.ShapeDtypeStruct((b, n, K), jnp.int32),
        ],
    )(xyz, xyz, xb, xb, sq_c, sq_r)
    return idx, cnt[:, :, 0], dst, bins


def _cls_body(y_ref, w1_ref, g1_ref, b1_ref, w2_ref, g2_ref, b2_ref,
              w3_ref, b3_ref, out_ref):
    y = y_ref[...]
    h = jnp.dot(y, w1_ref[...], preferred_element_type=jnp.float32)
    m = h.mean(axis=0, keepdims=True)
    v = ((h - m) ** 2).mean(axis=0, keepdims=True)
    h = (h - m) / jnp.sqrt(v + 1e-5) * g1_ref[...] + b1_ref[...]
    h = jnp.maximum(h, 0.0)
    h2 = jnp.dot(h, w2_ref[...], preferred_element_type=jnp.float32)
    m2 = h2.mean(axis=0, keepdims=True)
    v2 = ((h2 - m2) ** 2).mean(axis=0, keepdims=True)
    h2 = (h2 - m2) / jnp.sqrt(v2 + 1e-5) * g2_ref[...] + b2_ref[...]
    h2 = jnp.maximum(h2, 0.0)
    out_ref[...] = jnp.dot(h2, w3_ref[...], preferred_element_type=jnp.float32) + b3_ref[...]


def _cls_head(y, c):
    return pl.pallas_call(
        _cls_body,
        out_shape=jax.ShapeDtypeStruct((y.shape[0], c['W3'].shape[1]), jnp.float32),
    )(y, c['W1'], c['g1'][None, :], c['b1'][None, :],
      c['W2'], c['g2'][None, :], c['b2'][None, :],
      c['W3'], c['b3'][None, :])


def kernel(point_cloud, params):
    pc = jnp.transpose(point_cloud, (0, 2, 1))
    pc = pc - pc.mean(axis=1, keepdims=True)
    scale = jnp.sqrt(jnp.max(jnp.sum(pc * pc, axis=-1, keepdims=True), axis=1, keepdims=True))
    pc = pc / scale
    xyz = pc
    query = xyz.mean(axis=1, keepdims=True)
    net = jax.nn.relu(_bn(xyz @ params['fc1']['W'], params['fc1']['gamma'], params['fc1']['beta'], (0, 1)))
    global_feat = []
    index = 0
    for l in range(3):
        net = jnp.concatenate([net, xyz], axis=2)
        if l in _PALLAS_LAYERS:
            intra_idx, intra_cnt, intra_dst, filt_idx = _graph_pallas(xyz, RADIUS[l], NN_UP[l])
        else:
            intra_idx, intra_cnt, intra_dst, _ = _build_graph(xyz, RADIUS[l], NN_UP[l], NUM_SAMPLE[l])
            filt_idx = _spherical_kernel(xyz, xyz, intra_idx, intra_cnt, intra_dst, RADIUS[l], KERNEL)
        m_s = xyz.shape[1] // NUM_SAMPLE[l]
        indices = jnp.broadcast_to(jnp.arange(m_s) * NUM_SAMPLE[l], (xyz.shape[0], m_s))
        for _ in CHANNELS[l]:
            net = _sep_conv(net, intra_idx, intra_cnt, filt_idx, params['convs'][index])
            index += 1
        xyz = _bgather(xyz, indices)
        intra_idx = _bgather(intra_idx, indices)
        intra_cnt = _bgather(intra_cnt, indices)
        net = _pool3d(net, intra_idx, intra_cnt)
        global_feat.append(jnp.max(net, axis=1, keepdims=True))
    nl = xyz.shape[1]
    nn_idx = jnp.broadcast_to(jnp.arange(nl)[None, None, :], (xyz.shape[0], 1, nl))
    nn_cnt = jnp.full((xyz.shape[0], 1), nl, dtype=jnp.int32)
    nn_dst = jnp.sqrt(jnp.maximum(jnp.sum((xyz - query) ** 2, axis=-1), 1e-12))[:, None, :]
    filt_idx = _spherical_kernel(xyz, query, nn_idx, nn_cnt, nn_dst, GLOBAL_RADIUS, G_KERNEL)
    net = _sep_conv(net, nn_idx, nn_cnt, filt_idx, params['gconv'])
    global_feat.append(net)
    y = jnp.concatenate(global_feat, axis=2).reshape(point_cloud.shape[0], -1)
    return _cls_head(y, params['cls'])
